# Initial kernel scaffold; baseline (speedup 1.0000x reference)
import jax
import jax.numpy as jnp
from jax.experimental import pallas as pl


def kernel(x_pd, edge_index_pd, batch_pd, x_ast, edge_index_ast, batch_ast, x_cf, edge_index_cf, batch_cf, embed1, embed3, embed2, w1, gru1_wih, gru1_whh, gru1_bih, gru1_bhh, w2, gru2_wih, gru2_whh, gru2_bih, gru2_bhh, w3, gru3_wih, gru3_whh, gru3_bih, gru3_bhh, lin1_w, lin1_b, lin2_w, lin2_b, lin21_w, lin21_b, lin22_w, lin22_b, lin3_w, lin3_b, lin4_w, lin4_b):
    raise NotImplementedError("write your pallas kernel here")


# speedup vs baseline: 3.0033x; 3.0033x over previous
import functools

import jax
import jax.numpy as jnp
from jax import lax
from jax.experimental import pallas as pl
from jax.experimental.pallas import tpu as pltpu
from jax.experimental.pallas import tpu_sc as plsc

N = 10000
E = 160000
G = 64
H = 400
HQ = 100
SW = 128
ED = 20
NTOK = 20
NC = 2
NS = 16


def _sc_mesh():
    return plsc.VectorSubcoreMesh(core_axis_name="c", subcore_axis_name="s",
                                  num_cores=NC, num_subcores=NS)


_EMB_CHUNK = 80
_EMB_TOTAL = N * NTOK
_EMB_PER_CORE = _EMB_TOTAL // NC
_EMB_NCHUNK = _EMB_PER_CORE // _EMB_CHUNK


def _emb_sc_body(tok_hbm, table_hbm, out_hbm, idx_v, rows_v, sem):
    c = lax.axis_index("c")
    s = lax.axis_index("s")
    base = c * _EMB_PER_CORE

    def chunk(k, carry):
        q = s + k * NS
        @pl.when(q < _EMB_NCHUNK)
        def _():
            off = base + q * _EMB_CHUNK
            pltpu.sync_copy(tok_hbm.at[pl.ds(off, _EMB_CHUNK)], idx_v)
            pltpu.async_copy(table_hbm.at[idx_v], rows_v, sem).wait()
            pltpu.sync_copy(rows_v, out_hbm.at[pl.ds(off, _EMB_CHUNK)])
        return carry

    lax.fori_loop(0, (_EMB_NCHUNK + NS - 1) // NS, chunk, None)


@functools.cache
def _emb_sc():
    return pl.kernel(
        _emb_sc_body,
        out_type=jax.ShapeDtypeStruct((_EMB_TOTAL, SW), jnp.float32),
        mesh=_sc_mesh(),
        scratch_types=[
            pltpu.VMEM((_EMB_CHUNK,), jnp.int32),
            pltpu.VMEM((_EMB_CHUNK, SW), jnp.float32),
            pltpu.SemaphoreType.DMA,
        ],
        compiler_params=pltpu.CompilerParams(use_tc_tiling_on_sc=False),
    )


_SEG_CHUNK = 128
_SEG_NCHUNK = E // _SEG_CHUNK
_ROWS_PER_TILE = N // NS
_ZCHUNK = 125
_SEG_K = (_SEG_NCHUNK + NS - 1) // NS


def _segsum_sc_body(h4_hbm, src_hbm, dst_hbm, zeros_hbm, out_hbm,
                    acc_sh, sidxA, didxA, gidxA, sidxB, didxB, gidxB,
                    rowsA, rowsB, semA, semB):
    c = lax.axis_index("c")
    s = lax.axis_index("s")
    r0 = s * _ROWS_PER_TILE

    for p in range(2):
        slab = c * 2 + p
        nbase = slab * N
        for j in range(_ROWS_PER_TILE // _ZCHUNK):
            pltpu.sync_copy(zeros_hbm,
                            acc_sh.at[pl.ds(r0 + j * _ZCHUNK, _ZCHUNK)])
        plsc.subcore_barrier()

        def load_and_fire(q, sidx, didx, gidx, rows, sem):
            off = q * _SEG_CHUNK
            pltpu.sync_copy(src_hbm.at[pl.ds(off, _SEG_CHUNK)], sidx)
            pltpu.sync_copy(dst_hbm.at[pl.ds(off, _SEG_CHUNK)], didx)
            for j in range(_SEG_CHUNK // 16):
                sl = pl.ds(j * 16, 16)
                gidx[sl] = sidx[sl] + nbase
            return pltpu.async_copy(h4_hbm.at[gidx], rows, sem)

        @pl.when(s < _SEG_NCHUNK)
        def _():
            load_and_fire(s, sidxA, didxA, gidxA, rowsA, semA)

        def body(k2, carry):
            kA = 2 * k2
            kB = 2 * k2 + 1
            qA = s + kA * NS
            qB = s + kB * NS

            @pl.when(qB < _SEG_NCHUNK)
            def _():
                load_and_fire(qB, sidxB, didxB, gidxB, rowsB, semB)

            @pl.when(qA < _SEG_NCHUNK)
            def _():
                pltpu.make_async_copy(h4_hbm.at[gidxA], rowsA, semA).wait()
                pltpu.sync_copy(rowsA, acc_sh.at[didxA], add=True)

            qA2 = s + (kA + 2) * NS

            @pl.when(qA2 < _SEG_NCHUNK)
            def _():
                load_and_fire(qA2, sidxA, didxA, gidxA, rowsA, semA)

            @pl.when(qB < _SEG_NCHUNK)
            def _():
                pltpu.make_async_copy(h4_hbm.at[gidxB], rowsB, semB).wait()
                pltpu.sync_copy(rowsB, acc_sh.at[didxB], add=True)

            return carry

        lax.fori_loop(0, (_SEG_K + 1) // 2, body, None)
        plsc.subcore_barrier()
        for j in range(_ROWS_PER_TILE // _ZCHUNK):
            rr = r0 + j * _ZCHUNK
            pltpu.sync_copy(acc_sh.at[pl.ds(rr, _ZCHUNK)],
                            out_hbm.at[pl.ds(nbase + rr, _ZCHUNK)])


@functools.cache
def _segsum_sc():
    return pl.kernel(
        _segsum_sc_body,
        out_type=jax.ShapeDtypeStruct((4 * N, SW), jnp.float32),
        mesh=_sc_mesh(),
        scratch_types=[
            pltpu.VMEM_SHARED((N, SW), jnp.float32),
            pltpu.VMEM((_SEG_CHUNK,), jnp.int32),
            pltpu.VMEM((_SEG_CHUNK,), jnp.int32),
            pltpu.VMEM((_SEG_CHUNK,), jnp.int32),
            pltpu.VMEM((_SEG_CHUNK,), jnp.int32),
            pltpu.VMEM((_SEG_CHUNK,), jnp.int32),
            pltpu.VMEM((_SEG_CHUNK,), jnp.int32),
            pltpu.VMEM((_SEG_CHUNK, SW), jnp.float32),
            pltpu.VMEM((_SEG_CHUNK, SW), jnp.float32),
            pltpu.SemaphoreType.DMA,
            pltpu.SemaphoreType.DMA,
        ],
        compiler_params=pltpu.CompilerParams(use_tc_tiling_on_sc=False),
    )


_RBLK = 1000


def _repack_body(et_ref, o_ref):
    et = et_ref[...]
    zpad = jnp.zeros((_RBLK, SW - 5 * ED), jnp.float32)
    for q in range(4):
        parts = [et[:, 5 * q + t, :ED] for t in range(5)]
        o_ref[q] = jnp.concatenate(parts + [zpad], axis=1)


_repack_call = pl.pallas_call(
    _repack_body,
    grid=(N // _RBLK,),
    in_specs=[pl.BlockSpec((_RBLK, NTOK, SW), lambda i: (i, 0, 0))],
    out_specs=pl.BlockSpec((4, _RBLK, SW), lambda i: (0, i, 0)),
    out_shape=jax.ShapeDtypeStruct((4, N, SW), jnp.float32),
)


def _mslab_body(h_ref, w_ref, o_ref):
    h = jnp.concatenate([h_ref[q][:, :HQ] for q in range(4)], axis=1)
    m = lax.dot_general(h, w_ref[0], (((1,), (0,)), ((), ())),
                        preferred_element_type=jnp.float32)
    zpad = jnp.zeros((_BLK, SW - HQ), jnp.float32)
    for q in range(4):
        o_ref[q] = jnp.concatenate([m[:, q * HQ:(q + 1) * HQ], zpad], axis=1)


def _mslab_call(h4, w_l):
    return pl.pallas_call(
        _mslab_body,
        grid=(N // _BLK,),
        in_specs=[
            pl.BlockSpec((4, _BLK, SW), lambda i: (0, i, 0)),
            pl.BlockSpec((1, H, H), lambda i: (0, 0, 0)),
        ],
        out_specs=pl.BlockSpec((4, _BLK, SW), lambda i: (0, i, 0)),
        out_shape=jax.ShapeDtypeStruct((4, N, SW), jnp.float32),
        compiler_params=pltpu.CompilerParams(
            dimension_semantics=("arbitrary",)),
    )(h4, w_l)

_BLK = 1000


def _gru_body(agg_ref, h_ref, wih_ref, whh_ref, bih_ref, bhh_ref, out_ref):
    a = jnp.concatenate([agg_ref[q][:, :HQ] for q in range(4)], axis=1)
    h = jnp.concatenate([h_ref[q][:, :HQ] for q in range(4)], axis=1)
    gi = lax.dot_general(a, wih_ref[...], (((1,), (1,)), ((), ())),
                         preferred_element_type=jnp.float32) + bih_ref[...]
    gh = lax.dot_general(h, whh_ref[...], (((1,), (1,)), ((), ())),
                         preferred_element_type=jnp.float32) + bhh_ref[...]
    i_r, i_z, i_n = gi[:, :H], gi[:, H:2 * H], gi[:, 2 * H:]
    h_r, h_z, h_n = gh[:, :H], gh[:, H:2 * H], gh[:, 2 * H:]
    r = jax.nn.sigmoid(i_r + h_r)
    z = jax.nn.sigmoid(i_z + h_z)
    n = jnp.tanh(i_n + r * h_n)
    hn = (1.0 - z) * n + z * h
    zpad = jnp.zeros((_BLK, SW - HQ), jnp.float32)
    for q in range(4):
        out_ref[q] = jnp.concatenate([hn[:, q * HQ:(q + 1) * HQ], zpad],
                                     axis=1)


_gru_call = pl.pallas_call(
    _gru_body,
    grid=(N // _BLK,),
    in_specs=[
        pl.BlockSpec((4, _BLK, SW), lambda i: (0, i, 0)),
        pl.BlockSpec((4, _BLK, SW), lambda i: (0, i, 0)),
        pl.BlockSpec((3 * H, H), lambda i: (0, 0)),
        pl.BlockSpec((3 * H, H), lambda i: (0, 0)),
        pl.BlockSpec((1, 3 * H), lambda i: (0, 0)),
        pl.BlockSpec((1, 3 * H), lambda i: (0, 0)),
    ],
    out_specs=pl.BlockSpec((4, _BLK, SW), lambda i: (0, i, 0)),
    out_shape=jax.ShapeDtypeStruct((4, N, SW), jnp.float32),
    compiler_params=pltpu.CompilerParams(dimension_semantics=("arbitrary",)),
)

def _pool_body(batch_sm, h_ref, bcol_ref, o_ref):
    i = pl.program_id(0)

    @pl.when(i == 0)
    def _():
        o_ref[...] = jnp.zeros_like(o_ref)

    h = jnp.concatenate([h_ref[q][:, :HQ] for q in range(4)], axis=1)
    h = jnp.maximum(h, 0.0)
    bcol = bcol_ref[0]
    g_lo = batch_sm[i * _BLK]
    g_hi = batch_sm[i * _BLK + _BLK - 1]

    def body(g, carry):
        vals = jnp.where(bcol == g, h, 0.0)
        bm = jnp.max(vals, axis=0, keepdims=True)
        cur = o_ref[pl.ds(g, 1), :]
        o_ref[pl.ds(g, 1), :] = jnp.maximum(cur, bm)
        return carry

    lax.fori_loop(g_lo, g_hi + 1, body, None)


_pool_call = pl.pallas_call(
    _pool_body,
    grid_spec=pltpu.PrefetchScalarGridSpec(
        num_scalar_prefetch=1,
        grid=(N // _BLK,),
        in_specs=[
            pl.BlockSpec((4, _BLK, SW), lambda i, sm: (0, i, 0)),
            pl.BlockSpec((1, _BLK, 1), lambda i, sm: (i, 0, 0)),
        ],
        out_specs=pl.BlockSpec((G, H), lambda i, sm: (0, 0)),
    ),
    out_shape=jax.ShapeDtypeStruct((G, H), jnp.float32),
    compiler_params=pltpu.CompilerParams(dimension_semantics=("arbitrary",)),
)


def _head_body(p1, p2, p3, w1, b1, w2, b2, w21, b21, w22, b22, w3, b3, w4, b4,
               o_ref):
    x = jnp.concatenate([p1[...], p2[...], p3[...]], axis=1)
    for w, b in ((w1, b1), (w2, b2), (w21, b21), (w22, b22), (w3, b3),
                 (w4, b4)):
        x = lax.dot_general(x, w[...], (((1,), (1,)), ((), ())),
                            preferred_element_type=jnp.float32) + b[...]
        x = jnp.maximum(x, 0.0)
    o_ref[...] = x


def _head_call(p1, p2, p3, ws):
    return pl.pallas_call(
        _head_body,
        out_shape=jax.ShapeDtypeStruct((G, 4), jnp.float32),
    )(p1, p2, p3, *ws)


def _run_branch(tokens, edge_index, batch, table, w, wih, whh, bih, bhh,
                zeros_z):
    tok_flat = tokens.astype(jnp.int32).reshape(_EMB_TOTAL)
    table_pad = jnp.pad(table, ((0, 0), (0, SW - ED)))
    e_tok = _emb_sc()(tok_flat, table_pad)
    h = _repack_call(e_tok.reshape(N, NTOK, SW))

    src = edge_index[0].astype(jnp.int32)
    dst = edge_index[1].astype(jnp.int32)
    bih2 = bih.reshape(1, 3 * H)
    bhh2 = bhh.reshape(1, 3 * H)

    for l in range(3):
        m4 = _mslab_call(h, w[l:l + 1])
        agg4 = _segsum_sc()(m4.reshape(4 * N, SW), src, dst, zeros_z)
        h = _gru_call(agg4.reshape(4, N, SW), h, wih, whh, bih2, bhh2)

    batch = batch.astype(jnp.int32)
    pooled = _pool_call(batch, h, batch.reshape(N // _BLK, _BLK, 1))
    return pooled


def kernel(x_pd, edge_index_pd, batch_pd, x_ast, edge_index_ast, batch_ast,
           x_cf, edge_index_cf, batch_cf, embed1, embed3, embed2,
           w1, gru1_wih, gru1_whh, gru1_bih, gru1_bhh,
           w2, gru2_wih, gru2_whh, gru2_bih, gru2_bhh,
           w3, gru3_wih, gru3_whh, gru3_bih, gru3_bhh,
           lin1_w, lin1_b, lin2_w, lin2_b, lin21_w, lin21_b,
           lin22_w, lin22_b, lin3_w, lin3_b, lin4_w, lin4_b):
    zeros_z = jnp.zeros((_ZCHUNK, SW), jnp.float32)
    p1 = _run_branch(x_pd, edge_index_pd, batch_pd, embed1,
                     w1, gru1_wih, gru1_whh, gru1_bih, gru1_bhh, zeros_z)
    p2 = _run_branch(x_ast, edge_index_ast, batch_ast, embed3,
                     w2, gru2_wih, gru2_whh, gru2_bih, gru2_bhh, zeros_z)
    p3 = _run_branch(x_cf, edge_index_cf, batch_cf, embed2,
                     w3, gru3_wih, gru3_whh, gru3_bih, gru3_bhh, zeros_z)
    ws = (lin1_w, lin1_b.reshape(1, -1), lin2_w, lin2_b.reshape(1, -1),
          lin21_w, lin21_b.reshape(1, -1), lin22_w, lin22_b.reshape(1, -1),
          lin3_w, lin3_b.reshape(1, -1), lin4_w, lin4_b.reshape(1, -1))
    return _head_call(p1, p2, p3, ws)
